# baseline (speedup 1.0000x reference)
import functools

import jax
import jax.numpy as jnp
from jax import lax
from jax.experimental import pallas as pl
from jax.experimental import pallas as pl
from jax.experimental.pallas import tpu as pltpu
from jax.experimental.pallas import tpu_sc as plsc

BATCH = 16384
HIST_LEN = 50
EMBED_DIM = 64
VOCAB = 1000000

_info = plsc.get_sparse_core_info()
NC, NS = _info.num_cores, _info.num_subcores
NW = NC * NS
BPW = BATCH // NW
CB = BPW // 128
NBUF = CB


def _body(x_hbm, w_hbm, out_hbm, xs, cidx, pbuf, gbuf, tbuf, gsems, ssems):
    wid = lax.axis_index("s") * NC + lax.axis_index("c")
    pltpu.sync_copy(
        x_hbm.at[pl.ds(pl.multiple_of(wid * (BPW * HIST_LEN), 8), BPW * HIST_LEN)],
        xs)

    iota = lax.iota(jnp.int32, 16)

    def build_cidx(t, b):
        for v in range(8):
            flat = (iota + (128 * b + 16 * v)) * HIST_LEN + t
            raw = plsc.load_gather(xs, [flat])
            cidx[b, pl.ds(16 * v, 16)] = lax.shift_right_logical(raw, 1)
            pbuf[b, pl.ds(16 * v, 16)] = lax.shift_left(
                lax.bitwise_and(raw, 1), 6)

    def gather(b):
        return pltpu.make_async_copy(w_hbm.at[cidx.at[b]], gbuf.at[b], gsems.at[b])

    def store(t, b):
        cbg = wid * CB + b
        return pltpu.make_async_copy(tbuf.at[b], out_hbm.at[t, :, cbg], ssems.at[b])

    def transpose(b):
        pbs = [pbuf[b, pl.ds(16 * v, 16)] for v in range(8)]
        rows = [iota + 16 * v for v in range(8)]

        def eblock(e, carry):
            for f in range(8):
                for v in range(8):
                    cols = pbs[v] + (8 * e + f)
                    vals = plsc.load_gather(gbuf.at[b], [rows[v], cols])
                    tbuf[b, e, f, pl.ds(16 * v, 16)] = vals
            return carry
        lax.fori_loop(0, 8, eblock, 0)

    for b in range(NBUF):
        build_cidx(0, b)
        gather(b).start()

    def t_step(t, carry):
        for b in range(NBUF):
            gather(b).wait()

            @pl.when(t > 0)
            def _():
                store(t - 1, b).wait()

            transpose(b)
            store(t, b).start()

            @pl.when(t + 1 < HIST_LEN)
            def _():
                build_cidx(t + 1, b)
                gather(b).start()
        return carry

    lax.fori_loop(0, HIST_LEN, t_step, 0)
    for b in range(NBUF):
        store(HIST_LEN - 1, b).wait()


@jax.jit
def _embed(x, w2):
    mesh = plsc.VectorSubcoreMesh(core_axis_name="c", subcore_axis_name="s")
    return pl.kernel(
        _body,
        mesh=mesh,
        out_type=jax.ShapeDtypeStruct((HIST_LEN, 8, BATCH // 128, 8, 128), jnp.float32),
        scratch_types=[
            pltpu.VMEM((BPW * HIST_LEN,), jnp.int32),
            pltpu.VMEM((NBUF, 128), jnp.int32),
            pltpu.VMEM((NBUF, 128), jnp.int32),
            pltpu.VMEM((NBUF, 128, 128), jnp.float32),
            pltpu.VMEM((NBUF, 8, 8, 128), jnp.float32),
            pltpu.SemaphoreType.DMA((NBUF,)),
            pltpu.SemaphoreType.DMA((NBUF,)),
        ],
        compiler_params=pltpu.CompilerParams(
            use_tc_tiling_on_sc=False, needs_layout_passes=False),
    )(x.reshape(BATCH * HIST_LEN), w2)


def kernel(x, weight):
    w2 = weight.reshape(VOCAB // 2, 2 * EMBED_DIM)
    p = _embed(x.astype(jnp.int32), w2)
    return p.transpose(2, 4, 0, 1, 3).reshape(BATCH, HIST_LEN, EMBED_DIM)


# speedup vs baseline: 1.7509x; 1.7509x over previous
import functools

import jax
import jax.numpy as jnp
from jax import lax
from jax.experimental import pallas as pl
from jax.experimental.pallas import tpu as pltpu
from jax.experimental.pallas import tpu_sc as plsc

BATCH = 16384
HIST_LEN = 50
EMBED_DIM = 64
VOCAB = 1000000

_info = plsc.get_sparse_core_info()
NC, NS = _info.num_cores, _info.num_subcores
NW = NC * NS
BPW = BATCH // NW
CB = BPW // 128
NBUF = CB


def _body(x_hbm, w_hbm, out_hbm, xs, cidx, gbuf, tbuf, gsems, ssems):
    wid = lax.axis_index("s") * NC + lax.axis_index("c")
    pltpu.sync_copy(
        x_hbm.at[pl.ds(pl.multiple_of(wid * (BPW * HIST_LEN), 8), BPW * HIST_LEN)],
        xs)

    iota = lax.iota(jnp.int32, 16)
    pks = [lax.bitwise_and(iota + k, 15) for k in range(16)]
    e_of = [lax.shift_right_logical(iota + 16 * u, 3) for u in range(4)]
    f_of = lax.bitwise_and(iota, 7)
    cols_u = [iota + 16 * u for u in range(4)]

    def build_cidx(t, b):
        for v in range(8):
            flat = (iota + (128 * b + 16 * v)) * HIST_LEN + t
            cidx[b, pl.ds(16 * v, 16)] = plsc.load_gather(xs, [flat])

    def gather(b):
        return pltpu.make_async_copy(w_hbm.at[cidx.at[b]], gbuf.at[b], gsems.at[b])

    def store(t, b):
        cbg = wid * CB + b
        return pltpu.make_async_copy(tbuf.at[b], out_hbm.at[t, :, cbg], ssems.at[b])

    def transpose(b):
        def vblock(v, carry):
            for k in range(16):
                rowm = pks[k] + 16 * v
                for u in range(4):
                    vals = plsc.load_gather(gbuf.at[b], [rowm, cols_u[u]])
                    plsc.store_scatter(tbuf.at[b], [e_of[u], f_of, rowm], vals)
            return carry
        lax.fori_loop(0, 8, vblock, 0)

    for b in range(NBUF):
        build_cidx(0, b)
        gather(b).start()

    def t_step(t, carry):
        for b in range(NBUF):
            gather(b).wait()

            @pl.when(t > 0)
            def _():
                store(t - 1, b).wait()

            transpose(b)
            store(t, b).start()

            @pl.when(t + 1 < HIST_LEN)
            def _():
                build_cidx(t + 1, b)
                gather(b).start()
        return carry

    lax.fori_loop(0, HIST_LEN, t_step, 0)
    for b in range(NBUF):
        store(HIST_LEN - 1, b).wait()


@jax.jit
def _embed(x_flat, w):
    mesh = plsc.VectorSubcoreMesh(core_axis_name="c", subcore_axis_name="s")
    return pl.kernel(
        _body,
        mesh=mesh,
        out_type=jax.ShapeDtypeStruct(
            (HIST_LEN, 8, BATCH // 128, 8, 128), jnp.float32),
        scratch_types=[
            pltpu.VMEM((BPW * HIST_LEN,), jnp.int32),
            pltpu.VMEM((NBUF, 128), jnp.int32),
            pltpu.VMEM((NBUF, 128, EMBED_DIM), jnp.float32),
            pltpu.VMEM((NBUF, 8, 8, 128), jnp.float32),
            pltpu.SemaphoreType.DMA((NBUF,)),
            pltpu.SemaphoreType.DMA((NBUF,)),
        ],
        compiler_params=pltpu.CompilerParams(
            use_tc_tiling_on_sc=False, needs_layout_passes=False),
    )(x_flat, w)


def kernel(x, weight):
    p = _embed(x.reshape(BATCH * HIST_LEN).astype(jnp.int32), weight)
    return p.transpose(2, 4, 0, 1, 3).reshape(BATCH, HIST_LEN, EMBED_DIM)
